# Initial kernel scaffold; baseline (speedup 1.0000x reference)
import functools

import jax
import jax.numpy as jnp
from jax import lax
from jax.experimental import pallas as pl
from jax.experimental.pallas import tpu as pltpu
from jax.experimental.pallas import tpu_sc as plsc

N = 10000
D_IN = 128
HID = 256
NCLUS = 3

NCORE = 2
NSUB = 16
NW = NCORE * NSUB
B = 128
NB = 80
EPT = NB * B
EP = NW * EPT
NACC = 10240
DUMMY = N

_mesh = plsc.VectorSubcoreMesh(
    core_axis_name="c", subcore_axis_name="s", num_cores=NCORE,
    num_subcores=NSUB)


def _prep_body(src_hbm, dst_hbm, dstm_hbm, deg_hbm,
               src_v, dst_v, srcm_v, dstm_v, ones_v, zer_v, deg_sh, sem):
    c = lax.axis_index("c")
    s = lax.axis_index("s")
    wid = s * NCORE + c

    pltpu.sync_copy(src_hbm.at[wid], src_v)
    pltpu.sync_copy(dst_hbm.at[wid], dst_v)

    @pl.loop(0, B)
    def _(i):
        ones_v[i] = jnp.ones((16,), jnp.float32)

    @pl.loop(0, 40)
    def _(i):
        zer_v[i] = jnp.zeros((16,), jnp.float32)

    @pl.loop(0, 16)
    def _(k):
        pltpu.sync_copy(zer_v, deg_sh.at[pl.ds(s * 640 + k * 40, 40)])

    @pl.loop(0, NB)
    def _(j):
        for k in range(B // 16):
            sl = pl.ds(k * 16, 16)
            sv = src_v[j, sl]
            dv = dst_v[j, sl]
            loop = sv == dv
            srcm_v[j, sl] = jnp.where(loop, DUMMY, sv)
            dstm_v[j, sl] = jnp.where(loop, DUMMY, dv)

    pltpu.sync_copy(dstm_v, dstm_hbm.at[wid])
    plsc.subcore_barrier()

    @pl.loop(0, NB)
    def _(j):
        pltpu.sync_copy(ones_v, deg_sh.at[srcm_v.at[j]], add=True)

    plsc.subcore_barrier()
    pltpu.sync_copy(deg_sh.at[pl.ds(s * 640, 640)],
                    deg_hbm.at[c].at[pl.ds(s * 640, 640)])


_prep_call = functools.partial(
    pl.kernel,
    out_type=(
        jax.ShapeDtypeStruct((NW, NB, B), jnp.int32),
        jax.ShapeDtypeStruct((NCORE, NACC, 16), jnp.float32),
    ),
    mesh=_mesh,
    scratch_types=[
        pltpu.VMEM((NB, B), jnp.int32),
        pltpu.VMEM((NB, B), jnp.int32),
        pltpu.VMEM((NB, B), jnp.int32),
        pltpu.VMEM((NB, B), jnp.int32),
        pltpu.VMEM((B, 16), jnp.float32),
        pltpu.VMEM((40, 16), jnp.float32),
        pltpu.VMEM_SHARED((NACC, 16), jnp.float32),
        pltpu.SemaphoreType.DMA,
    ],
)(_prep_body)


def _make_agg(dc):
    def body(hn_hbm, srcg_hbm, dstm_hbm, acc_hbm,
             src_v, dst_v, rows_v, acc_sh, sem):
        c = lax.axis_index("c")
        s = lax.axis_index("s")
        wid = s * NCORE + c

        pltpu.sync_copy(srcg_hbm.at[wid], src_v)
        pltpu.sync_copy(dstm_hbm.at[wid], dst_v)

        @pl.when(s < 15)
        def _():
            pltpu.sync_copy(hn_hbm.at[c].at[pl.ds(s * 640, 640)],
                            acc_sh.at[pl.ds(s * 640, 640)])

        @pl.when(s == 15)
        def _():
            pltpu.sync_copy(hn_hbm.at[c].at[pl.ds(9600, 400)],
                            acc_sh.at[pl.ds(9600, 400)])

        plsc.subcore_barrier()

        @pl.loop(0, NB)
        def _(j):
            pltpu.async_copy(hn_hbm.at[c].at[src_v.at[j]], rows_v, sem).wait()
            pltpu.sync_copy(rows_v, acc_sh.at[dst_v.at[j]], add=True)

        plsc.subcore_barrier()
        pltpu.sync_copy(acc_sh.at[pl.ds(s * 625, 625)],
                        acc_hbm.at[c].at[pl.ds(s * 625, 625)])

    return functools.partial(
        pl.kernel,
        out_type=jax.ShapeDtypeStruct((NCORE, N, dc), jnp.float32),
        mesh=_mesh,
        scratch_types=[
            pltpu.VMEM((NB, B), jnp.int32),
            pltpu.VMEM((NB, B), jnp.int32),
            pltpu.VMEM((B, dc), jnp.float32),
            pltpu.VMEM_SHARED((NACC, dc), jnp.float32),
            pltpu.SemaphoreType.DMA,
        ],
    )(body)


_agg64 = _make_agg(64)
_agg128 = _make_agg(128)


def _tcprep_body(deg_ref, x0_ref, x1_ref, invd_ref, h0_ref, h1_ref):
    deg = deg_ref[0, :N, 0] + deg_ref[1, :N, 0] + 1.0
    invd = 1.0 / jnp.maximum(deg, 1.0)
    invd_ref[...] = invd[:, None]
    for xr, hr in ((x0_ref, h0_ref), (x1_ref, h1_ref)):
        hn = xr[...] * invd[:, None]
        hr[0] = hn[:, :64]
        hr[1] = hn[:, 64:]


def _tc_prep(deg, x0, x1):
    return pl.pallas_call(
        _tcprep_body,
        out_shape=(
            jax.ShapeDtypeStruct((N, 1), jnp.float32),
            jax.ShapeDtypeStruct((2, N, 64), jnp.float32),
            jax.ShapeDtypeStruct((2, N, 64), jnp.float32),
        ),
    )(deg, x0, x1)


_BN = 1000


def _tclayer_body(acc_ref, w_ref, b_ref, a_ref, invd_ref, out_ref):
    agg = jnp.concatenate([acc_ref[0], acc_ref[1]], axis=1)
    h = jnp.dot(agg, w_ref[...], preferred_element_type=jnp.float32)
    h = h + b_ref[...]
    a = a_ref[0, 0]
    h = jnp.where(h >= 0, h, a * h)
    hn = h * invd_ref[...]
    half = out_ref.shape[2]
    out_ref[0] = hn[:, :half]
    out_ref[1] = hn[:, half:]


def _tc_layer(acc, w, b, a, invd, dc_out):
    d_in = 2 * acc.shape[2]
    d_out = w.shape[1]
    grid = (N // _BN,)
    return pl.pallas_call(
        _tclayer_body,
        grid=grid,
        in_specs=[
            pl.BlockSpec((2, _BN, acc.shape[2]), lambda i: (0, i, 0)),
            pl.BlockSpec((d_in, d_out), lambda i: (0, 0)),
            pl.BlockSpec((1, d_out), lambda i: (0, 0)),
            pl.BlockSpec((1, 1), lambda i: (0, 0)),
            pl.BlockSpec((_BN, 1), lambda i: (i, 0)),
        ],
        out_specs=pl.BlockSpec((2, _BN, dc_out), lambda i: (0, i, 0)),
        out_shape=jax.ShapeDtypeStruct((2, N, dc_out), jnp.float32),
    )(acc, w, b.reshape(1, -1), a.reshape(1, 1), invd)


def _tcfinal_body(acc_ref, w_ref, b_ref, a_ref,
                  iw0_ref, ib0_ref, iw1_ref, ib1_ref, ai_ref,
                  cw0_ref, cb0_ref, cw1_ref, cb1_ref, ac_ref,
                  z_ref, c_ref):
    agg = jnp.concatenate([acc_ref[0], acc_ref[1]], axis=1)
    g = jnp.dot(agg, w_ref[...], preferred_element_type=jnp.float32)
    g = g + b_ref[...]
    a = a_ref[0, 0]
    g = jnp.where(g >= 0, g, a * g)

    ai = ai_ref[0, 0]
    t = jnp.dot(g, iw0_ref[...], preferred_element_type=jnp.float32) + ib0_ref[...]
    t = jnp.where(t >= 0, t, ai * t)
    z_ref[...] = jnp.dot(t, iw1_ref[...], preferred_element_type=jnp.float32) + ib1_ref[...]

    ac = ac_ref[0, 0]
    u = jnp.dot(g, cw0_ref[...], preferred_element_type=jnp.float32) + cb0_ref[...]
    u = jnp.where(u >= 0, u, ac * u)
    logits = jnp.dot(u, cw1_ref[...], preferred_element_type=jnp.float32) + cb1_ref[...]
    m = jnp.max(logits, axis=1, keepdims=True)
    e = jnp.exp(logits - m)
    p = e / jnp.sum(e, axis=1, keepdims=True)
    nrm = jnp.sqrt(jnp.sum(p * p, axis=1, keepdims=True))
    c_ref[...] = p / jnp.maximum(nrm, 1e-12)


def _tc_final(acc, w, b, a, pi, pc):
    cw1 = jnp.pad(pc['W1'], ((0, 0), (0, 128 - NCLUS)))
    cb1 = jnp.pad(pc['b1'], (0, 128 - NCLUS), constant_values=-1e30)
    grid = (N // _BN,)
    full = lambda r, c_: pl.BlockSpec((r, c_), lambda i: (0, 0))
    return pl.pallas_call(
        _tcfinal_body,
        grid=grid,
        in_specs=[
            pl.BlockSpec((2, _BN, 128), lambda i: (0, i, 0)),
            full(HID, HID), full(1, HID), full(1, 1),
            full(HID, HID), full(1, HID), full(HID, HID), full(1, HID),
            full(1, 1),
            full(HID, HID), full(1, HID), full(HID, 128), full(1, 128),
            full(1, 1),
        ],
        out_specs=(
            pl.BlockSpec((_BN, HID), lambda i: (i, 0)),
            pl.BlockSpec((_BN, 128), lambda i: (i, 0)),
        ),
        out_shape=(
            jax.ShapeDtypeStruct((N, HID), jnp.float32),
            jax.ShapeDtypeStruct((N, 128), jnp.float32),
        ),
    )(acc, w, b.reshape(1, -1), a.reshape(1, 1),
      pi['W0'], pi['b0'].reshape(1, -1), pi['W1'], pi['b1'].reshape(1, -1),
      pi['a'].reshape(1, 1),
      pc['W0'], pc['b0'].reshape(1, -1), cw1, cb1.reshape(1, -1),
      pc['a'].reshape(1, 1))


def kernel(x0, x1, params, edge_index):
    src = edge_index[0]
    dst = edge_index[1]
    e = src.shape[0]
    pad = EP - e
    src_p = jnp.concatenate([src, jnp.zeros((pad,), jnp.int32)]).reshape(NW, NB, B)
    dst_p = jnp.concatenate([dst, jnp.zeros((pad,), jnp.int32)]).reshape(NW, NB, B)

    dstm, deg = _prep_call(src_p, dst_p)
    invd, h0_v0, h0_v1 = _tc_prep(deg, x0, x1)

    outs_z = []
    outs_c = []
    for v, h0 in ((0, h0_v0), (1, h0_v1)):
        p = params[v]
        gw, gb, ga = p['g']['W'], p['g']['b'], p['g']['a']
        acc1 = _agg64(h0, src_p, dstm)
        hn1 = _tc_layer(acc1, gw[0], gb[0], ga, invd, 128)
        acc2 = _agg128(hn1, src_p, dstm)
        hn2 = _tc_layer(acc2, gw[1], gb[1], ga, invd, 128)
        acc3 = _agg128(hn2, src_p, dstm)
        z, cpad = _tc_final(acc3, gw[2], gb[2], ga, p['inst'], p['clus'])
        outs_z.append(z)
        outs_c.append(cpad[:, :NCLUS])

    zs = tuple(outs_z)
    cs = tuple(outs_c)
    return (zs, zs, zs, cs, cs)


# speedup vs baseline: 1.8375x; 1.8375x over previous
import functools

import jax
import jax.numpy as jnp
from jax import lax
from jax.experimental import pallas as pl
from jax.experimental.pallas import tpu as pltpu
from jax.experimental.pallas import tpu_sc as plsc

N = 10000
D_IN = 128
HID = 256
NCLUS = 3

NCORE = 2
NSUB = 16
NW = NCORE * NSUB
B = 128
NB = 80
NB2 = 160
EPT = NB * B
EP = NW * EPT
NACC = 10240
DUMMY = N

_mesh = plsc.VectorSubcoreMesh(
    core_axis_name="c", subcore_axis_name="s", num_cores=NCORE,
    num_subcores=NSUB)


def _slice_copy(s, src_at, dst_at):
    @pl.when(s < 15)
    def _():
        pltpu.sync_copy(src_at(pl.ds(s * 640, 640)), dst_at(pl.ds(s * 640, 640)))

    @pl.when(s == 15)
    def _():
        pltpu.sync_copy(src_at(pl.ds(9600, 400)), dst_at(pl.ds(9600, 400)))


def _mask_body(src_hbm, dst_hbm, srcm_hbm, dstm_hbm,
               src_v, dst_v, srcm_v, dstm_v, sem):
    c = lax.axis_index("c")
    s = lax.axis_index("s")
    wid = s * NCORE + c

    pltpu.sync_copy(src_hbm.at[wid], src_v)
    pltpu.sync_copy(dst_hbm.at[wid], dst_v)

    @pl.loop(0, NB)
    def _(j):
        for k in range(B // 16):
            sl = pl.ds(k * 16, 16)
            sv = src_v[j, sl]
            dv = dst_v[j, sl]
            loop = sv == dv
            srcm_v[j, sl] = jnp.where(loop, DUMMY, sv)
            dstm_v[j, sl] = jnp.where(loop, DUMMY, dv)

    pltpu.sync_copy(srcm_v, srcm_hbm.at[wid])
    pltpu.sync_copy(dstm_v, dstm_hbm.at[wid])


_mask_call = functools.partial(
    pl.kernel,
    out_type=(
        jax.ShapeDtypeStruct((NW, NB, B), jnp.int32),
        jax.ShapeDtypeStruct((NW, NB, B), jnp.int32),
    ),
    mesh=_mesh,
    scratch_types=[
        pltpu.VMEM((NB, B), jnp.int32),
        pltpu.VMEM((NB, B), jnp.int32),
        pltpu.VMEM((NB, B), jnp.int32),
        pltpu.VMEM((NB, B), jnp.int32),
        pltpu.SemaphoreType.DMA,
    ],
)(_mask_body)


def _deg_body(srcm_hbm, ones_hbm, zero_hbm, deg_hbm,
              src_v, ones_v, deg_sh, sem):
    c = lax.axis_index("c")
    s = lax.axis_index("s")
    wid = s * NCORE + c

    pltpu.sync_copy(srcm_hbm.at[wid], src_v)
    pltpu.sync_copy(ones_hbm, ones_v)
    _slice_copy(s, lambda d: zero_hbm.at[d], lambda d: deg_sh.at[d])
    plsc.subcore_barrier()

    @pl.loop(0, NB)
    def _(j):
        pltpu.sync_copy(ones_v, deg_sh.at[src_v.at[j]], add=True)

    plsc.subcore_barrier()
    _slice_copy(s, lambda d: deg_sh.at[d], lambda d: deg_hbm.at[c].at[d])


_deg_call = functools.partial(
    pl.kernel,
    out_type=jax.ShapeDtypeStruct((NCORE, N, 128), jnp.float32),
    mesh=_mesh,
    scratch_types=[
        pltpu.VMEM((NB, B), jnp.int32),
        pltpu.VMEM((B, 128), jnp.float32),
        pltpu.VMEM_SHARED((NACC, 128), jnp.float32),
        pltpu.SemaphoreType.DMA,
    ],
)(_deg_body)


def _aggl1_body(hn_hbm, zero_hbm, srcg_hbm, dstm_hbm, acc_hbm,
                src_v, dst_v, rows_v, acc_sh, sem):
    c = lax.axis_index("c")
    s = lax.axis_index("s")
    wid = c * NSUB + s

    pltpu.sync_copy(srcg_hbm.at[wid], src_v)
    pltpu.sync_copy(dstm_hbm.at[wid], dst_v)

    @pl.when(c == 0)
    def _():
        _slice_copy(s, lambda d: hn_hbm.at[d], lambda d: acc_sh.at[d])

    @pl.when(c == 1)
    def _():
        _slice_copy(s, lambda d: zero_hbm.at[d], lambda d: acc_sh.at[d])

    plsc.subcore_barrier()

    @pl.loop(0, NB)
    def _(j):
        pltpu.async_copy(hn_hbm.at[src_v.at[j]], rows_v, sem).wait()
        pltpu.sync_copy(rows_v, acc_sh.at[dst_v.at[j]], add=True)

    plsc.subcore_barrier()
    _slice_copy(s, lambda d: acc_sh.at[d], lambda d: acc_hbm.at[c].at[d])


_agg_l1 = functools.partial(
    pl.kernel,
    out_type=jax.ShapeDtypeStruct((NCORE, N, 128), jnp.float32),
    mesh=_mesh,
    scratch_types=[
        pltpu.VMEM((NB, B), jnp.int32),
        pltpu.VMEM((NB, B), jnp.int32),
        pltpu.VMEM((B, 128), jnp.float32),
        pltpu.VMEM_SHARED((NACC, 128), jnp.float32),
        pltpu.SemaphoreType.DMA,
    ],
)(_aggl1_body)


def _tcprep_body(deg_ref, x0_ref, x1_ref, invd_ref, h0_ref, h1_ref):
    deg = deg_ref[0, :, 0] + deg_ref[1, :, 0] + 1.0
    invd = 1.0 / jnp.maximum(deg, 1.0)
    invd_ref[...] = invd[:, None]
    h0_ref[...] = x0_ref[...] * invd[:, None]
    h1_ref[...] = x1_ref[...] * invd[:, None]


def _tc_prep(deg, x0, x1):
    return pl.pallas_call(
        _tcprep_body,
        out_shape=(
            jax.ShapeDtypeStruct((N, 1), jnp.float32),
            jax.ShapeDtypeStruct((N, D_IN), jnp.float32),
            jax.ShapeDtypeStruct((N, D_IN), jnp.float32),
        ),
    )(deg, x0, x1)


_BN = 1000


def _tclayer_body(nacc, acc_refs, w_ref, b_ref, a_ref, invd_ref, out_ref):
    agg = jnp.concatenate([r[0] + r[1] for r in acc_refs], axis=1)
    h = jnp.dot(agg, w_ref[...], preferred_element_type=jnp.float32)
    h = h + b_ref[...]
    a = a_ref[0, 0]
    h = jnp.where(h >= 0, h, a * h)
    hn = h * invd_ref[...]
    half = out_ref.shape[2]
    out_ref[0] = hn[:, :half]
    out_ref[1] = hn[:, half:]


def _tc_layer(accs, w, b, a, invd):
    nacc = len(accs)
    d_in = w.shape[0]
    d_out = w.shape[1]
    dc_out = d_out // 2
    grid = (N // _BN,)

    def body(*refs):
        acc_refs = refs[:nacc]
        w_ref, b_ref, a_ref, invd_ref, out_ref = refs[nacc:]
        _tclayer_body(nacc, acc_refs, w_ref, b_ref, a_ref, invd_ref, out_ref)

    return pl.pallas_call(
        body,
        grid=grid,
        in_specs=[pl.BlockSpec((2, _BN, 128), lambda i: (0, i, 0))] * nacc + [
            pl.BlockSpec((d_in, d_out), lambda i: (0, 0)),
            pl.BlockSpec((1, d_out), lambda i: (0, 0)),
            pl.BlockSpec((1, 1), lambda i: (0, 0)),
            pl.BlockSpec((_BN, 1), lambda i: (i, 0)),
        ],
        out_specs=pl.BlockSpec((2, _BN, dc_out), lambda i: (0, i, 0)),
        out_shape=jax.ShapeDtypeStruct((2, N, dc_out), jnp.float32),
    )(*accs, w, b.reshape(1, -1), a.reshape(1, 1), invd)


def _tcfinal_body(acca_ref, accb_ref, w_ref, b_ref, a_ref,
                  iw0_ref, ib0_ref, iw1_ref, ib1_ref, ai_ref,
                  cw0_ref, cb0_ref, cw1_ref, cb1_ref, ac_ref,
                  z_ref, c_ref):
    agg = jnp.concatenate([acca_ref[0] + acca_ref[1],
                           accb_ref[0] + accb_ref[1]], axis=1)
    g = jnp.dot(agg, w_ref[...], preferred_element_type=jnp.float32)
    g = g + b_ref[...]
    a = a_ref[0, 0]
    g = jnp.where(g >= 0, g, a * g)

    ai = ai_ref[0, 0]
    t = jnp.dot(g, iw0_ref[...], preferred_element_type=jnp.float32) + ib0_ref[...]
    t = jnp.where(t >= 0, t, ai * t)
    z_ref[...] = jnp.dot(t, iw1_ref[...], preferred_element_type=jnp.float32) + ib1_ref[...]

    ac = ac_ref[0, 0]
    u = jnp.dot(g, cw0_ref[...], preferred_element_type=jnp.float32) + cb0_ref[...]
    u = jnp.where(u >= 0, u, ac * u)
    logits = jnp.dot(u, cw1_ref[...], preferred_element_type=jnp.float32) + cb1_ref[...]
    m = jnp.max(logits, axis=1, keepdims=True)
    e = jnp.exp(logits - m)
    p = e / jnp.sum(e, axis=1, keepdims=True)
    nrm = jnp.sqrt(jnp.sum(p * p, axis=1, keepdims=True))
    c_ref[...] = p / jnp.maximum(nrm, 1e-12)


def _tc_final(acca, accb, w, b, a, pi, pc):
    cw1 = jnp.pad(pc['W1'], ((0, 0), (0, 128 - NCLUS)))
    cb1 = jnp.pad(pc['b1'], (0, 128 - NCLUS), constant_values=-1e30)
    grid = (N // _BN,)
    full = lambda r, c_: pl.BlockSpec((r, c_), lambda i: (0, 0))
    return pl.pallas_call(
        _tcfinal_body,
        grid=grid,
        in_specs=[
            pl.BlockSpec((2, _BN, 128), lambda i: (0, i, 0)),
            pl.BlockSpec((2, _BN, 128), lambda i: (0, i, 0)),
            full(HID, HID), full(1, HID), full(1, 1),
            full(HID, HID), full(1, HID), full(HID, HID), full(1, HID),
            full(1, 1),
            full(HID, HID), full(1, HID), full(HID, 128), full(1, 128),
            full(1, 1),
        ],
        out_specs=(
            pl.BlockSpec((_BN, HID), lambda i: (i, 0)),
            pl.BlockSpec((_BN, 128), lambda i: (i, 0)),
        ),
        out_shape=(
            jax.ShapeDtypeStruct((N, HID), jnp.float32),
            jax.ShapeDtypeStruct((N, 128), jnp.float32),
        ),
    )(acca, accb, w, b.reshape(1, -1), a.reshape(1, 1),
      pi['W0'], pi['b0'].reshape(1, -1), pi['W1'], pi['b1'].reshape(1, -1),
      pi['a'].reshape(1, 1),
      pc['W0'], pc['b0'].reshape(1, -1), cw1, cb1.reshape(1, -1),
      pc['a'].reshape(1, 1))


def kernel(x0, x1, params, edge_index):
    src = edge_index[0]
    dst = edge_index[1]
    e = src.shape[0]
    pad = EP - e
    src_p = jnp.concatenate([src, jnp.zeros((pad,), jnp.int32)]).reshape(NW, NB, B)
    dst_p = jnp.concatenate([dst, jnp.zeros((pad,), jnp.int32)]).reshape(NW, NB, B)

    srcm, dstm = _mask_call(src_p, dst_p)
    zero_hbm = jnp.zeros((N, D_IN), jnp.float32)
    ones_hbm = jnp.ones((B, 128), jnp.float32)
    deg = _deg_call(srcm, ones_hbm, zero_hbm)
    invd, h0_v0, h0_v1 = _tc_prep(deg, x0, x1)

    outs_z = []
    outs_c = []
    for v, h0 in ((0, h0_v0), (1, h0_v1)):
        p = params[v]
        gw, gb, ga = p['g']['W'], p['g']['b'], p['g']['a']
        acc1 = _agg_l1(h0, zero_hbm, src_p, dstm)
        hn1 = _tc_layer([acc1], gw[0], gb[0], ga, invd)
        acc2a = _agg_l1(hn1[0], zero_hbm, src_p, dstm)
        acc2b = _agg_l1(hn1[1], zero_hbm, src_p, dstm)
        hn2 = _tc_layer([acc2a, acc2b], gw[1], gb[1], ga, invd)
        acc3a = _agg_l1(hn2[0], zero_hbm, src_p, dstm)
        acc3b = _agg_l1(hn2[1], zero_hbm, src_p, dstm)
        z, cpad = _tc_final(acc3a, acc3b, gw[2], gb[2], ga, p['inst'], p['clus'])
        outs_z.append(z)
        outs_c.append(cpad[:, :NCLUS])

    zs = tuple(outs_z)
    cs = tuple(outs_c)
    return (zs, zs, zs, cs, cs)


# speedup vs baseline: 2.1888x; 1.1912x over previous
import functools

import jax
import jax.numpy as jnp
from jax import lax
from jax.experimental import pallas as pl
from jax.experimental.pallas import tpu as pltpu
from jax.experimental.pallas import tpu_sc as plsc

N = 10000
D_IN = 128
HID = 256
NCLUS = 3

NCORE = 2
NSUB = 16
NW = NCORE * NSUB
B = 128
NB = 80
NPH = 2
NBP = NB // NPH
EPT = NB * B
EP = NW * EPT
NACC = 10240
NACCA = N + 8
DUMMY = N

_mesh = plsc.VectorSubcoreMesh(
    core_axis_name="c", subcore_axis_name="s", num_cores=NCORE,
    num_subcores=NSUB)


def _slice_copy(s, src_at, dst_at):
    @pl.when(s < 15)
    def _():
        pltpu.sync_copy(src_at(pl.ds(s * 640, 640)), dst_at(pl.ds(s * 640, 640)))

    @pl.when(s == 15)
    def _():
        pltpu.sync_copy(src_at(pl.ds(9600, 400)), dst_at(pl.ds(9600, 400)))


def _mask_body(src_hbm, dst_hbm, srcm_hbm, dstm_hbm,
               src_v, dst_v, srcm_v, dstm_v, sem):
    c = lax.axis_index("c")
    s = lax.axis_index("s")
    wid = s * NCORE + c

    pltpu.sync_copy(src_hbm.at[wid], src_v)
    pltpu.sync_copy(dst_hbm.at[wid], dst_v)

    @pl.loop(0, NB)
    def _(j):
        for k in range(B // 16):
            sl = pl.ds(k * 16, 16)
            sv = src_v[j, sl]
            dv = dst_v[j, sl]
            keep = sv != dv
            srcm_v[j, sl] = jnp.where(keep, sv, DUMMY)
            dstm_v[j, sl] = jnp.where(keep, dv, DUMMY)

    pltpu.sync_copy(srcm_v, srcm_hbm.at[wid])
    pltpu.sync_copy(dstm_v, dstm_hbm.at[wid])


_mask_call = functools.partial(
    pl.kernel,
    out_type=(
        jax.ShapeDtypeStruct((NW, NB, B), jnp.int32),
        jax.ShapeDtypeStruct((NW, NB, B), jnp.int32),
    ),
    mesh=_mesh,
    scratch_types=[pltpu.VMEM((NB, B), jnp.int32)] * 4 + [
        pltpu.SemaphoreType.DMA,
    ],
)(_mask_body)


def _deg_body(srcm_hbm, ones_hbm, zero_hbm, deg_hbm,
              src_v, ones_v, deg_sh, sem):
    c = lax.axis_index("c")
    s = lax.axis_index("s")
    wid = s * NCORE + c

    pltpu.sync_copy(srcm_hbm.at[wid], src_v)
    pltpu.sync_copy(ones_hbm, ones_v)
    _slice_copy(s, lambda d: zero_hbm.at[d], lambda d: deg_sh.at[d])
    plsc.subcore_barrier()

    @pl.loop(0, NB)
    def _(j):
        pltpu.sync_copy(ones_v, deg_sh.at[src_v.at[j]], add=True)

    plsc.subcore_barrier()
    _slice_copy(s, lambda d: deg_sh.at[d], lambda d: deg_hbm.at[c].at[d])


_deg_call = functools.partial(
    pl.kernel,
    out_type=jax.ShapeDtypeStruct((NCORE, N, 128), jnp.float32),
    mesh=_mesh,
    scratch_types=[
        pltpu.VMEM((NB, B), jnp.int32),
        pltpu.VMEM((B, 128), jnp.float32),
        pltpu.VMEM_SHARED((NACC, 128), jnp.float32),
        pltpu.SemaphoreType.DMA,
    ],
)(_deg_body)


def _agg_body(hn_hbm, zero_hbm, srcg_hbm, dstm_hbm, acc_hbm,
              src_v, dst_v, rows0, rows1, g0, g1, s0, s1, acc_sh):
    rows = (rows0, rows1)
    gsem = (g0, g1)
    ssem = (s0, s1)
    c = lax.axis_index("c")
    s = lax.axis_index("s")
    wid = c * NSUB + s

    @pl.when(c == 0)
    def _():
        _slice_copy(s, lambda d: hn_hbm.at[d], lambda d: acc_sh.at[d])

    @pl.when(c == 1)
    def _():
        _slice_copy(s, lambda d: zero_hbm.at[d], lambda d: acc_sh.at[d])

    plsc.subcore_barrier()

    for ph in range(NPH):
        base = wid * NB + ph * NBP
        pltpu.sync_copy(srcg_hbm.at[pl.ds(base, NBP)], src_v)
        pltpu.sync_copy(dstm_hbm.at[pl.ds(base, NBP)], dst_v)

        def start_g(j, b):
            pltpu.async_copy(hn_hbm.at[src_v.at[j]], rows[b], gsem[b])

        start_g(0, 0)
        start_g(1, 1)

        @pl.loop(0, NBP, step=2)
        def _(jj):
            for b in range(2):
                j = jj + b
                pltpu.make_async_copy(hn_hbm.at[src_v.at[j]], rows[b],
                                      gsem[b]).wait()
                pltpu.async_copy(rows[b], acc_sh.at[dst_v.at[j]], ssem[b],
                                 add=True)

                @pl.when(j + 2 < NBP)
                def _():
                    pltpu.make_async_copy(rows[b], acc_sh.at[dst_v.at[j]],
                                          ssem[b]).wait()
                    start_g(j + 2, b)

        for i in range(2):
            j = NBP - 2 + i
            pltpu.make_async_copy(rows[j % 2], acc_sh.at[dst_v.at[j]],
                                  ssem[j % 2]).wait()

    plsc.subcore_barrier()
    _slice_copy(s, lambda d: acc_sh.at[d], lambda d: acc_hbm.at[c].at[d])


_agg_call = functools.partial(
    pl.kernel,
    out_type=jax.ShapeDtypeStruct((NCORE, N, 128), jnp.float32),
    mesh=_mesh,
    scratch_types=[
        pltpu.VMEM((NBP, B), jnp.int32),
        pltpu.VMEM((NBP, B), jnp.int32),
        pltpu.VMEM((B, 128), jnp.float32),
        pltpu.VMEM((B, 128), jnp.float32),
    ] + [pltpu.SemaphoreType.DMA] * 4 + [
        pltpu.VMEM_SHARED((NACCA, 128), jnp.float32),
    ],
)(_agg_body)


def _tcprep_body(deg_ref, x0_ref, x1_ref, invd_ref, h0_ref, h1_ref):
    deg = deg_ref[0, :, 0] + deg_ref[1, :, 0] + 1.0
    invd = 1.0 / jnp.maximum(deg, 1.0)
    invd_ref[...] = invd[:, None]
    h0_ref[...] = x0_ref[...] * invd[:, None]
    h1_ref[...] = x1_ref[...] * invd[:, None]


def _tc_prep(deg, x0, x1):
    return pl.pallas_call(
        _tcprep_body,
        out_shape=(
            jax.ShapeDtypeStruct((N, 1), jnp.float32),
            jax.ShapeDtypeStruct((N, D_IN), jnp.float32),
            jax.ShapeDtypeStruct((N, D_IN), jnp.float32),
        ),
    )(deg, x0, x1)


_BN = 1000


def _tc_layer(accs, w, b, a, invd):
    nacc = len(accs)
    d_in = w.shape[0]
    d_out = w.shape[1]
    dc_out = d_out // 2
    grid = (N // _BN,)

    def body(*refs):
        acc_refs = refs[:nacc]
        w_ref, b_ref, a_ref, invd_ref, out_ref = refs[nacc:]
        parts = [r[0] + r[1] for r in acc_refs]
        agg = parts[0] if nacc == 1 else jnp.concatenate(parts, axis=1)
        h = jnp.dot(agg, w_ref[...], preferred_element_type=jnp.float32)
        h = h + b_ref[...]
        av = a_ref[0, 0]
        h = jnp.where(h >= 0, h, av * h)
        hn = h * invd_ref[...]
        out_ref[0] = hn[:, :dc_out]
        out_ref[1] = hn[:, dc_out:]

    return pl.pallas_call(
        body,
        grid=grid,
        in_specs=[pl.BlockSpec((2, _BN, 128), lambda i: (0, i, 0))] * nacc + [
            pl.BlockSpec((d_in, d_out), lambda i: (0, 0)),
            pl.BlockSpec((1, d_out), lambda i: (0, 0)),
            pl.BlockSpec((1, 1), lambda i: (0, 0)),
            pl.BlockSpec((_BN, 1), lambda i: (i, 0)),
        ],
        out_specs=pl.BlockSpec((2, _BN, dc_out), lambda i: (0, i, 0)),
        out_shape=jax.ShapeDtypeStruct((2, N, dc_out), jnp.float32),
    )(*accs, w, b.reshape(1, -1), a.reshape(1, 1), invd)


def _tcfinal_body(acca_ref, accb_ref, w_ref, b_ref, a_ref,
                  iw0_ref, ib0_ref, iw1_ref, ib1_ref, ai_ref,
                  cw0_ref, cb0_ref, cw1_ref, cb1_ref, ac_ref,
                  z_ref, c_ref):
    agg = jnp.concatenate([acca_ref[0] + acca_ref[1],
                           accb_ref[0] + accb_ref[1]], axis=1)
    g = jnp.dot(agg, w_ref[...], preferred_element_type=jnp.float32)
    g = g + b_ref[...]
    a = a_ref[0, 0]
    g = jnp.where(g >= 0, g, a * g)

    ai = ai_ref[0, 0]
    t = jnp.dot(g, iw0_ref[...], preferred_element_type=jnp.float32) + ib0_ref[...]
    t = jnp.where(t >= 0, t, ai * t)
    z_ref[...] = jnp.dot(t, iw1_ref[...], preferred_element_type=jnp.float32) + ib1_ref[...]

    ac = ac_ref[0, 0]
    u = jnp.dot(g, cw0_ref[...], preferred_element_type=jnp.float32) + cb0_ref[...]
    u = jnp.where(u >= 0, u, ac * u)
    logits = jnp.dot(u, cw1_ref[...], preferred_element_type=jnp.float32) + cb1_ref[...]
    m = jnp.max(logits, axis=1, keepdims=True)
    e = jnp.exp(logits - m)
    p = e / jnp.sum(e, axis=1, keepdims=True)
    nrm = jnp.sqrt(jnp.sum(p * p, axis=1, keepdims=True))
    c_ref[...] = p / jnp.maximum(nrm, 1e-12)


def _tc_final(acca, accb, w, b, a, pi, pc):
    cw1 = jnp.pad(pc['W1'], ((0, 0), (0, 128 - NCLUS)))
    cb1 = jnp.pad(pc['b1'], (0, 128 - NCLUS), constant_values=-1e30)
    grid = (N // _BN,)
    full = lambda r, c_: pl.BlockSpec((r, c_), lambda i: (0, 0))
    return pl.pallas_call(
        _tcfinal_body,
        grid=grid,
        in_specs=[
            pl.BlockSpec((2, _BN, 128), lambda i: (0, i, 0)),
            pl.BlockSpec((2, _BN, 128), lambda i: (0, i, 0)),
            full(HID, HID), full(1, HID), full(1, 1),
            full(HID, HID), full(1, HID), full(HID, HID), full(1, HID),
            full(1, 1),
            full(HID, HID), full(1, HID), full(HID, 128), full(1, 128),
            full(1, 1),
        ],
        out_specs=(
            pl.BlockSpec((_BN, HID), lambda i: (i, 0)),
            pl.BlockSpec((_BN, 128), lambda i: (i, 0)),
        ),
        out_shape=(
            jax.ShapeDtypeStruct((N, HID), jnp.float32),
            jax.ShapeDtypeStruct((N, 128), jnp.float32),
        ),
    )(acca, accb, w, b.reshape(1, -1), a.reshape(1, 1),
      pi['W0'], pi['b0'].reshape(1, -1), pi['W1'], pi['b1'].reshape(1, -1),
      pi['a'].reshape(1, 1),
      pc['W0'], pc['b0'].reshape(1, -1), cw1, cb1.reshape(1, -1),
      pc['a'].reshape(1, 1))


def kernel(x0, x1, params, edge_index):
    src = edge_index[0]
    dst = edge_index[1]
    e = src.shape[0]
    pad = EP - e
    src_p = jnp.concatenate([src, jnp.zeros((pad,), jnp.int32)]).reshape(NW, NB, B)
    dst_p = jnp.concatenate([dst, jnp.zeros((pad,), jnp.int32)]).reshape(NW, NB, B)

    srcm, dstm = _mask_call(src_p, dst_p)
    zero_hbm = jnp.zeros((N, D_IN), jnp.float32)
    ones_hbm = jnp.ones((B, 128), jnp.float32)
    deg = _deg_call(srcm, ones_hbm, zero_hbm)
    invd, h0_v0, h0_v1 = _tc_prep(deg, x0, x1)
    srca = src_p.reshape(NW * NB, B)
    dstma = dstm.reshape(NW * NB, B)

    def agg(hn):
        return _agg_call(hn, zero_hbm, srca, dstma)

    outs_z = []
    outs_c = []
    for v, h0 in ((0, h0_v0), (1, h0_v1)):
        p = params[v]
        gw, gb, ga = p['g']['W'], p['g']['b'], p['g']['a']
        acc1 = agg(h0)
        hn1 = _tc_layer([acc1], gw[0], gb[0], ga, invd)
        acc2a = agg(hn1[0])
        acc2b = agg(hn1[1])
        hn2 = _tc_layer([acc2a, acc2b], gw[1], gb[1], ga, invd)
        acc3a = agg(hn2[0])
        acc3b = agg(hn2[1])
        z, cpad = _tc_final(acc3a, acc3b, gw[2], gb[2], ga, p['inst'], p['clus'])
        outs_z.append(z)
        outs_c.append(cpad[:, :NCLUS])

    zs = tuple(outs_z)
    cs = tuple(outs_c)
    return (zs, zs, zs, cs, cs)


# speedup vs baseline: 2.2035x; 1.0067x over previous
import functools

import jax
import jax.numpy as jnp
from jax import lax
from jax.experimental import pallas as pl
from jax.experimental.pallas import tpu as pltpu
from jax.experimental.pallas import tpu_sc as plsc

N = 10000
D_IN = 128
HID = 256
NCLUS = 3

NCORE = 2
NSUB = 16
NW = NCORE * NSUB
B = 128
NB = 80
NPH = 2
NBP = NB // NPH
EPT = NB * B
EP = NW * EPT
NACC = 10240
NACCA = N + 8
DUMMY = N

_mesh = plsc.VectorSubcoreMesh(
    core_axis_name="c", subcore_axis_name="s", num_cores=NCORE,
    num_subcores=NSUB)


def _slice_copy(s, src_at, dst_at):
    @pl.when(s < 15)
    def _():
        pltpu.sync_copy(src_at(pl.ds(s * 640, 640)), dst_at(pl.ds(s * 640, 640)))

    @pl.when(s == 15)
    def _():
        pltpu.sync_copy(src_at(pl.ds(9600, 400)), dst_at(pl.ds(9600, 400)))


def _mask_body(src_hbm, dst_hbm, srcm_hbm, dstm_hbm,
               src_v, dst_v, srcm_v, dstm_v, sem):
    c = lax.axis_index("c")
    s = lax.axis_index("s")
    wid = s * NCORE + c

    pltpu.sync_copy(src_hbm.at[wid], src_v)
    pltpu.sync_copy(dst_hbm.at[wid], dst_v)

    @pl.loop(0, NB)
    def _(j):
        for k in range(B // 16):
            sl = pl.ds(k * 16, 16)
            sv = src_v[j, sl]
            dv = dst_v[j, sl]
            keep = sv != dv
            srcm_v[j, sl] = jnp.where(keep, sv, DUMMY)
            dstm_v[j, sl] = jnp.where(keep, dv, DUMMY)

    pltpu.sync_copy(srcm_v, srcm_hbm.at[wid])
    pltpu.sync_copy(dstm_v, dstm_hbm.at[wid])


_mask_call = functools.partial(
    pl.kernel,
    out_type=(
        jax.ShapeDtypeStruct((NW, NB, B), jnp.int32),
        jax.ShapeDtypeStruct((NW, NB, B), jnp.int32),
    ),
    mesh=_mesh,
    scratch_types=[pltpu.VMEM((NB, B), jnp.int32)] * 4 + [
        pltpu.SemaphoreType.DMA,
    ],
)(_mask_body)


def _deg_body(srcm_hbm, ones_hbm, zero_hbm, deg_hbm,
              src_v, ones_v, deg_sh, sem):
    c = lax.axis_index("c")
    s = lax.axis_index("s")
    wid = s * NCORE + c

    pltpu.sync_copy(srcm_hbm.at[wid], src_v)
    pltpu.sync_copy(ones_hbm, ones_v)
    _slice_copy(s, lambda d: zero_hbm.at[d], lambda d: deg_sh.at[d])
    plsc.subcore_barrier()

    @pl.loop(0, NB)
    def _(j):
        pltpu.sync_copy(ones_v, deg_sh.at[src_v.at[j]], add=True)

    plsc.subcore_barrier()
    _slice_copy(s, lambda d: deg_sh.at[d], lambda d: deg_hbm.at[c].at[d])


_deg_call = functools.partial(
    pl.kernel,
    out_type=jax.ShapeDtypeStruct((NCORE, N, 128), jnp.float32),
    mesh=_mesh,
    scratch_types=[
        pltpu.VMEM((NB, B), jnp.int32),
        pltpu.VMEM((B, 128), jnp.float32),
        pltpu.VMEM_SHARED((NACC, 128), jnp.float32),
        pltpu.SemaphoreType.DMA,
    ],
)(_deg_body)


def _agg_body(hn_hbm, zero_hbm, srcg_hbm, dstm_hbm, acc_hbm,
              src_v, dst_v, rows0, rows1, g0, g1, s0, s1, acc_sh):
    rows = (rows0, rows1)
    gsem = (g0, g1)
    ssem = (s0, s1)
    c = lax.axis_index("c")
    s = lax.axis_index("s")
    wid = c * NSUB + s

    @pl.when(c == 0)
    def _():
        _slice_copy(s, lambda d: hn_hbm.at[d], lambda d: acc_sh.at[d])

    @pl.when(c == 1)
    def _():
        _slice_copy(s, lambda d: zero_hbm.at[d], lambda d: acc_sh.at[d])

    plsc.subcore_barrier()

    for ph in range(NPH):
        base = wid * NB + ph * NBP
        pltpu.sync_copy(srcg_hbm.at[pl.ds(base, NBP)], src_v)
        pltpu.sync_copy(dstm_hbm.at[pl.ds(base, NBP)], dst_v)

        def start_g(j, b):
            pltpu.async_copy(hn_hbm.at[src_v.at[j].at[pl.ds(0, 64)]],
                             rows[b].at[pl.ds(0, 64)], gsem[b])
            pltpu.async_copy(hn_hbm.at[src_v.at[j].at[pl.ds(64, 64)]],
                             rows[b].at[pl.ds(64, 64)], gsem[b])

        def wait_g(j, b):
            pltpu.make_async_copy(hn_hbm.at[src_v.at[j].at[pl.ds(0, 64)]],
                                  rows[b].at[pl.ds(0, 64)], gsem[b]).wait()
            pltpu.make_async_copy(hn_hbm.at[src_v.at[j].at[pl.ds(64, 64)]],
                                  rows[b].at[pl.ds(64, 64)], gsem[b]).wait()

        start_g(0, 0)
        start_g(1, 1)

        @pl.loop(0, NBP, step=2)
        def _(jj):
            for b in range(2):
                j = jj + b
                wait_g(j, b)
                pltpu.async_copy(rows[b], acc_sh.at[dst_v.at[j]], ssem[b],
                                 add=True)

                @pl.when(j + 2 < NBP)
                def _():
                    pltpu.make_async_copy(rows[b], acc_sh.at[dst_v.at[j]],
                                          ssem[b]).wait()
                    start_g(j + 2, b)

        for i in range(2):
            j = NBP - 2 + i
            pltpu.make_async_copy(rows[j % 2], acc_sh.at[dst_v.at[j]],
                                  ssem[j % 2]).wait()

    plsc.subcore_barrier()
    _slice_copy(s, lambda d: acc_sh.at[d], lambda d: acc_hbm.at[c].at[d])


_agg_call = functools.partial(
    pl.kernel,
    out_type=jax.ShapeDtypeStruct((NCORE, N, 128), jnp.float32),
    mesh=_mesh,
    scratch_types=[
        pltpu.VMEM((NBP, B), jnp.int32),
        pltpu.VMEM((NBP, B), jnp.int32),
        pltpu.VMEM((B, 128), jnp.float32),
        pltpu.VMEM((B, 128), jnp.float32),
    ] + [pltpu.SemaphoreType.DMA] * 4 + [
        pltpu.VMEM_SHARED((NACCA, 128), jnp.float32),
    ],
)(_agg_body)


def _tcprep_body(deg_ref, x0_ref, x1_ref, invd_ref, h0_ref, h1_ref):
    deg = deg_ref[0, :, 0] + deg_ref[1, :, 0] + 1.0
    invd = 1.0 / jnp.maximum(deg, 1.0)
    invd_ref[...] = invd[:, None]
    h0_ref[...] = x0_ref[...] * invd[:, None]
    h1_ref[...] = x1_ref[...] * invd[:, None]


def _tc_prep(deg, x0, x1):
    return pl.pallas_call(
        _tcprep_body,
        out_shape=(
            jax.ShapeDtypeStruct((N, 1), jnp.float32),
            jax.ShapeDtypeStruct((N, D_IN), jnp.float32),
            jax.ShapeDtypeStruct((N, D_IN), jnp.float32),
        ),
    )(deg, x0, x1)


_BN = 1000


def _tc_layer(accs, w, b, a, invd):
    nacc = len(accs)
    d_in = w.shape[0]
    d_out = w.shape[1]
    dc_out = d_out // 2
    grid = (N // _BN,)

    def body(*refs):
        acc_refs = refs[:nacc]
        w_ref, b_ref, a_ref, invd_ref, out_ref = refs[nacc:]
        parts = [r[0] + r[1] for r in acc_refs]
        agg = parts[0] if nacc == 1 else jnp.concatenate(parts, axis=1)
        h = jnp.dot(agg, w_ref[...], preferred_element_type=jnp.float32)
        h = h + b_ref[...]
        av = a_ref[0, 0]
        h = jnp.where(h >= 0, h, av * h)
        hn = h * invd_ref[...]
        out_ref[0] = hn[:, :dc_out]
        out_ref[1] = hn[:, dc_out:]

    return pl.pallas_call(
        body,
        grid=grid,
        in_specs=[pl.BlockSpec((2, _BN, 128), lambda i: (0, i, 0))] * nacc + [
            pl.BlockSpec((d_in, d_out), lambda i: (0, 0)),
            pl.BlockSpec((1, d_out), lambda i: (0, 0)),
            pl.BlockSpec((1, 1), lambda i: (0, 0)),
            pl.BlockSpec((_BN, 1), lambda i: (i, 0)),
        ],
        out_specs=pl.BlockSpec((2, _BN, dc_out), lambda i: (0, i, 0)),
        out_shape=jax.ShapeDtypeStruct((2, N, dc_out), jnp.float32),
    )(*accs, w, b.reshape(1, -1), a.reshape(1, 1), invd)


def _tcfinal_body(acca_ref, accb_ref, w_ref, b_ref, a_ref,
                  iw0_ref, ib0_ref, iw1_ref, ib1_ref, ai_ref,
                  cw0_ref, cb0_ref, cw1_ref, cb1_ref, ac_ref,
                  z_ref, c_ref):
    agg = jnp.concatenate([acca_ref[0] + acca_ref[1],
                           accb_ref[0] + accb_ref[1]], axis=1)
    g = jnp.dot(agg, w_ref[...], preferred_element_type=jnp.float32)
    g = g + b_ref[...]
    a = a_ref[0, 0]
    g = jnp.where(g >= 0, g, a * g)

    ai = ai_ref[0, 0]
    t = jnp.dot(g, iw0_ref[...], preferred_element_type=jnp.float32) + ib0_ref[...]
    t = jnp.where(t >= 0, t, ai * t)
    z_ref[...] = jnp.dot(t, iw1_ref[...], preferred_element_type=jnp.float32) + ib1_ref[...]

    ac = ac_ref[0, 0]
    u = jnp.dot(g, cw0_ref[...], preferred_element_type=jnp.float32) + cb0_ref[...]
    u = jnp.where(u >= 0, u, ac * u)
    logits = jnp.dot(u, cw1_ref[...], preferred_element_type=jnp.float32) + cb1_ref[...]
    m = jnp.max(logits, axis=1, keepdims=True)
    e = jnp.exp(logits - m)
    p = e / jnp.sum(e, axis=1, keepdims=True)
    nrm = jnp.sqrt(jnp.sum(p * p, axis=1, keepdims=True))
    c_ref[...] = p / jnp.maximum(nrm, 1e-12)


def _tc_final(acca, accb, w, b, a, pi, pc):
    cw1 = jnp.pad(pc['W1'], ((0, 0), (0, 128 - NCLUS)))
    cb1 = jnp.pad(pc['b1'], (0, 128 - NCLUS), constant_values=-1e30)
    grid = (N // _BN,)
    full = lambda r, c_: pl.BlockSpec((r, c_), lambda i: (0, 0))
    return pl.pallas_call(
        _tcfinal_body,
        grid=grid,
        in_specs=[
            pl.BlockSpec((2, _BN, 128), lambda i: (0, i, 0)),
            pl.BlockSpec((2, _BN, 128), lambda i: (0, i, 0)),
            full(HID, HID), full(1, HID), full(1, 1),
            full(HID, HID), full(1, HID), full(HID, HID), full(1, HID),
            full(1, 1),
            full(HID, HID), full(1, HID), full(HID, 128), full(1, 128),
            full(1, 1),
        ],
        out_specs=(
            pl.BlockSpec((_BN, HID), lambda i: (i, 0)),
            pl.BlockSpec((_BN, 128), lambda i: (i, 0)),
        ),
        out_shape=(
            jax.ShapeDtypeStruct((N, HID), jnp.float32),
            jax.ShapeDtypeStruct((N, 128), jnp.float32),
        ),
    )(acca, accb, w, b.reshape(1, -1), a.reshape(1, 1),
      pi['W0'], pi['b0'].reshape(1, -1), pi['W1'], pi['b1'].reshape(1, -1),
      pi['a'].reshape(1, 1),
      pc['W0'], pc['b0'].reshape(1, -1), cw1, cb1.reshape(1, -1),
      pc['a'].reshape(1, 1))


def kernel(x0, x1, params, edge_index):
    src = edge_index[0]
    dst = edge_index[1]
    e = src.shape[0]
    pad = EP - e
    src_p = jnp.concatenate([src, jnp.zeros((pad,), jnp.int32)]).reshape(NW, NB, B)
    dst_p = jnp.concatenate([dst, jnp.zeros((pad,), jnp.int32)]).reshape(NW, NB, B)

    srcm, dstm = _mask_call(src_p, dst_p)
    zero_hbm = jnp.zeros((N, D_IN), jnp.float32)
    ones_hbm = jnp.ones((B, 128), jnp.float32)
    deg = _deg_call(srcm, ones_hbm, zero_hbm)
    invd, h0_v0, h0_v1 = _tc_prep(deg, x0, x1)
    srca = src_p.reshape(NW * NB, B)
    dstma = dstm.reshape(NW * NB, B)

    def agg(hn):
        return _agg_call(hn, zero_hbm, srca, dstma)

    outs_z = []
    outs_c = []
    for v, h0 in ((0, h0_v0), (1, h0_v1)):
        p = params[v]
        gw, gb, ga = p['g']['W'], p['g']['b'], p['g']['a']
        acc1 = agg(h0)
        hn1 = _tc_layer([acc1], gw[0], gb[0], ga, invd)
        acc2a = agg(hn1[0])
        acc2b = agg(hn1[1])
        hn2 = _tc_layer([acc2a, acc2b], gw[1], gb[1], ga, invd)
        acc3a = agg(hn2[0])
        acc3b = agg(hn2[1])
        z, cpad = _tc_final(acc3a, acc3b, gw[2], gb[2], ga, p['inst'], p['clus'])
        outs_z.append(z)
        outs_c.append(cpad[:, :NCLUS])

    zs = tuple(outs_z)
    cs = tuple(outs_c)
    return (zs, zs, zs, cs, cs)
